# baseline (speedup 1.0000x reference)
import functools

import jax
import jax.numpy as jnp
from jax import lax
from jax.experimental import pallas as pl
from jax.experimental.pallas import tpu as pltpu
from jax.experimental.pallas import tpu_sc as plsc

T, DH, DHID, E, TOPK = 8192, 1024, 2048, 8, 2

BT = 256
BM = 256
NPAD = T * TOPK + E * BM
NB = NPAD // BM

NC, NS = 2, 16
NW = NC * NS


def _sc_mesh():
    return plsc.VectorSubcoreMesh(core_axis_name="c", subcore_axis_name="s")


def _wid():
    return lax.axis_index("s") * NC + lax.axis_index("c")


_G_ROWS = NPAD // NW
_G_CH = 32
_G_NCH = _G_ROWS // _G_CH


def _gather_body(x_hbm, tok_hbm, xs_hbm, idx0, idx1, buf0, buf1, sem0, sem1):
    base = _wid() * _G_ROWS
    idxs = (idx0, idx1)
    bufs = (buf0, buf1)
    sems = (sem0, sem1)
    copies = [None, None]
    for c in range(_G_NCH):
        s = c % 2
        if copies[s] is not None:
            copies[s].wait()
            pltpu.sync_copy(bufs[s], xs_hbm.at[pl.ds(base + (c - 2) * _G_CH,
                                                     _G_CH)])
        pltpu.sync_copy(tok_hbm.at[pl.ds(base + c * _G_CH, _G_CH)], idxs[s])
        copies[s] = pltpu.async_copy(x_hbm.at[idxs[s]], bufs[s], sems[s])
    for c in range(_G_NCH, _G_NCH + 2):
        s = c % 2
        copies[s].wait()
        pltpu.sync_copy(bufs[s], xs_hbm.at[pl.ds(base + (c - 2) * _G_CH,
                                                 _G_CH)])


def _sc_gather(x, tok_pad):
    k = pl.kernel(
        _gather_body,
        out_type=jax.ShapeDtypeStruct((NPAD, DH // 2), jnp.int32),
        mesh=_sc_mesh(),
        scratch_types=[
            pltpu.VMEM((_G_CH,), jnp.int32),
            pltpu.VMEM((_G_CH,), jnp.int32),
            pltpu.VMEM((_G_CH, DH // 2), jnp.int32),
            pltpu.VMEM((_G_CH, DH // 2), jnp.int32),
            pltpu.SemaphoreType.DMA,
            pltpu.SemaphoreType.DMA,
        ],
    )
    return k(x, tok_pad)


_C_TOK = T // NW
_C_CH = 32
_C_NCH = _C_TOK // _C_CH


def _combine_body(ys_hbm, s0_hbm, s1_hbm, y_hbm,
                  i0_v, i1_v, bufa, bufb, sema, semb):
    base = _wid() * _C_TOK
    for c in range(_C_NCH):
        off = base + c * _C_CH
        pltpu.sync_copy(s0_hbm.at[pl.ds(off, _C_CH)], i0_v)
        pltpu.sync_copy(s1_hbm.at[pl.ds(off, _C_CH)], i1_v)
        cpa = pltpu.async_copy(ys_hbm.at[i0_v], bufa, sema)
        cpb = pltpu.async_copy(ys_hbm.at[i1_v], bufb, semb)
        cpa.wait()
        cpb.wait()

        def add_row(r, _):
            for j in range(DH // 16):
                sl = pl.ds(j * 16, 16)
                bufa[r, sl] = bufa[r, sl] + bufb[r, sl]
            return 0

        lax.fori_loop(0, _C_CH, add_row, 0)
        pltpu.sync_copy(bufa, y_hbm.at[pl.ds(off, _C_CH)])


def _sc_combine(ys, src0, src1):
    k = pl.kernel(
        _combine_body,
        out_type=jax.ShapeDtypeStruct((T, DH), jnp.float32),
        mesh=_sc_mesh(),
        scratch_types=[
            pltpu.VMEM((_C_CH,), jnp.int32),
            pltpu.VMEM((_C_CH,), jnp.int32),
            pltpu.VMEM((_C_CH, DH), jnp.float32),
            pltpu.VMEM((_C_CH, DH), jnp.float32),
            pltpu.SemaphoreType.DMA,
            pltpu.SemaphoreType.DMA,
        ],
    )
    return k(ys, src0, src1)


def _router_body(x_ref, wr_ref, a0_ref, a1_ref, w0_ref, w1_ref):
    xb = x_ref[...]
    logits = jax.lax.dot_general(
        xb, wr_ref[...], (((1,), (1,)), ((), ())),
        preferred_element_type=jnp.float32)
    iota = jax.lax.broadcasted_iota(jnp.int32, logits.shape, 1)
    m1 = jnp.max(logits, axis=1, keepdims=True)
    a0 = jnp.min(jnp.where(logits == m1, iota, E), axis=1, keepdims=True)
    masked = jnp.where(iota == a0, -jnp.inf, logits)
    m2 = jnp.max(masked, axis=1, keepdims=True)
    a1 = jnp.min(jnp.where(masked == m2, iota, E), axis=1, keepdims=True)
    r = jnp.exp(m2 - m1)
    w0 = 1.0 / (1.0 + r)
    a0_ref[...] = a0
    a1_ref[...] = a1
    w0_ref[...] = w0
    w1_ref[...] = r * w0


def _router(x, Wr):
    nt = T // BT
    return pl.pallas_call(
        _router_body,
        grid=(nt,),
        in_specs=[
            pl.BlockSpec((BT, DH), lambda i: (i, 0)),
            pl.BlockSpec((E, DH), lambda i: (0, 0)),
        ],
        out_specs=[
            pl.BlockSpec((BT, 1), lambda i: (i, 0)),
            pl.BlockSpec((BT, 1), lambda i: (i, 0)),
            pl.BlockSpec((BT, 1), lambda i: (i, 0)),
            pl.BlockSpec((BT, 1), lambda i: (i, 0)),
        ],
        out_shape=[
            jax.ShapeDtypeStruct((T, 1), jnp.int32),
            jax.ShapeDtypeStruct((T, 1), jnp.int32),
            jax.ShapeDtypeStruct((T, 1), jnp.float32),
            jax.ShapeDtypeStruct((T, 1), jnp.float32),
        ],
    )(x, Wr)


def _ffn_body(be_ref, xs_ref, win_ref, wout_ref, wp_ref, ys_ref):
    xb = xs_ref[...]
    h = jax.lax.dot_general(
        xb, win_ref[0], (((1,), (1,)), ((), ())),
        preferred_element_type=jnp.float32)
    h = 0.5 * h * (1.0 + jax.lax.erf(h * 0.7071067811865476))
    out = jax.lax.dot_general(
        h.astype(jnp.bfloat16), wout_ref[0], (((1,), (0,)), ((), ())),
        preferred_element_type=jnp.float32)
    ys_ref[...] = out * wp_ref[...]


def _grouped_ffn(xs, W_in, W_out, w_pad, blk_exp):
    grid_spec = pltpu.PrefetchScalarGridSpec(
        num_scalar_prefetch=1,
        grid=(NB,),
        in_specs=[
            pl.BlockSpec((BM, DH), lambda b, be: (b, 0)),
            pl.BlockSpec((1, DHID, DH), lambda b, be: (be[b], 0, 0)),
            pl.BlockSpec((1, DHID, DH), lambda b, be: (be[b], 0, 0)),
            pl.BlockSpec((BM, 1), lambda b, be: (b, 0)),
        ],
        out_specs=pl.BlockSpec((BM, DH), lambda b, be: (b, 0)),
    )
    return pl.pallas_call(
        _ffn_body,
        grid_spec=grid_spec,
        out_shape=jax.ShapeDtypeStruct((NPAD, DH), jnp.float32),
    )(blk_exp, xs, W_in, W_out, w_pad)


def kernel(x, Wr, W_in, W_out):
    a0, a1, w0, w1 = _router(x, Wr)

    e_flat = jnp.concatenate([a0, a1], axis=1).reshape(-1)
    w_flat = jnp.concatenate([w0, w1], axis=1).reshape(-1)
    t_flat = jnp.arange(T * TOPK, dtype=jnp.int32) // TOPK
    oh = (e_flat[:, None] == jnp.arange(E, dtype=jnp.int32)[None, :])
    csum = jnp.cumsum(oh.astype(jnp.int32), axis=0)
    rank = jnp.take_along_axis(csum, e_flat[:, None], axis=1)[:, 0] - 1
    counts = csum[-1]
    padded = ((counts + BM - 1) // BM) * BM
    ends = jnp.cumsum(padded)
    off = ends - padded
    dst = off[e_flat] + rank

    tok_pad = jnp.zeros((NPAD,), jnp.int32).at[dst].set(
        t_flat, unique_indices=True)
    w_pad = jnp.zeros((NPAD, 1), jnp.float32).at[dst, 0].set(
        w_flat, unique_indices=True)
    blk_starts = jnp.arange(NB, dtype=jnp.int32) * BM
    blk_exp = jnp.minimum(
        jnp.sum((blk_starts[:, None] >= ends[None, :]).astype(jnp.int32),
                axis=1), E - 1)

    xi = jax.lax.bitcast_convert_type(
        x.astype(jnp.bfloat16).reshape(T, DH // 2, 2), jnp.int32)
    xsi = _sc_gather(xi, tok_pad)
    xs = jax.lax.bitcast_convert_type(
        xsi, jnp.bfloat16).reshape(NPAD, DH)

    ys = _grouped_ffn(xs, W_in.astype(jnp.bfloat16),
                      W_out.astype(jnp.bfloat16), w_pad, blk_exp)

    src = dst.reshape(T, TOPK)
    y = _sc_combine(ys, src[:, 0], src[:, 1])
    return y


# speedup vs baseline: 1.8708x; 1.8708x over previous
import functools

import jax
import jax.numpy as jnp
from jax import lax
from jax.experimental import pallas as pl
from jax.experimental.pallas import tpu as pltpu
from jax.experimental.pallas import tpu_sc as plsc

T, DH, DHID, E, TOPK = 8192, 1024, 2048, 8, 2

BT = 256
BM = 256
NPAD = T * TOPK + E * BM
NB = NPAD // BM

NC, NS = 2, 16
NW = NC * NS


def _sc_mesh():
    return plsc.VectorSubcoreMesh(core_axis_name="c", subcore_axis_name="s")


def _wid():
    return lax.axis_index("s") * NC + lax.axis_index("c")


_G_ROWS = NPAD // NW
_G_CH = 32
_G_NCH = _G_ROWS // _G_CH


def _gather_body(x_hbm, tok_hbm, xs_hbm, idx0, idx1, buf0, buf1, sem0, sem1):
    base = _wid() * _G_ROWS
    idxs = (idx0, idx1)
    bufs = (buf0, buf1)
    sems = (sem0, sem1)
    copies = [None, None]
    for c in range(_G_NCH):
        s = c % 2
        if copies[s] is not None:
            copies[s].wait()
            pltpu.sync_copy(bufs[s], xs_hbm.at[pl.ds(base + (c - 2) * _G_CH,
                                                     _G_CH)])
        pltpu.sync_copy(tok_hbm.at[pl.ds(base + c * _G_CH, _G_CH)], idxs[s])
        copies[s] = pltpu.async_copy(x_hbm.at[idxs[s]], bufs[s], sems[s])
    for c in range(_G_NCH, _G_NCH + 2):
        s = c % 2
        copies[s].wait()
        pltpu.sync_copy(bufs[s], xs_hbm.at[pl.ds(base + (c - 2) * _G_CH,
                                                 _G_CH)])


def _sc_gather(x, tok_pad):
    k = pl.kernel(
        _gather_body,
        out_type=jax.ShapeDtypeStruct((NPAD, DH), jnp.float32),
        mesh=_sc_mesh(),
        scratch_types=[
            pltpu.VMEM((_G_CH,), jnp.int32),
            pltpu.VMEM((_G_CH,), jnp.int32),
            pltpu.VMEM((_G_CH, DH), jnp.float32),
            pltpu.VMEM((_G_CH, DH), jnp.float32),
            pltpu.SemaphoreType.DMA,
            pltpu.SemaphoreType.DMA,
        ],
    )
    return k(x, tok_pad)


_C_TOK = T // NW
_C_CH = 32
_C_NCH = _C_TOK // _C_CH


def _combine_body(ys_hbm, s0_hbm, s1_hbm, y_hbm,
                  i0_v, i1_v, bufa, bufb, sema, semb):
    base = _wid() * _C_TOK
    for c in range(_C_NCH):
        off = base + c * _C_CH
        pltpu.sync_copy(s0_hbm.at[pl.ds(off, _C_CH)], i0_v)
        pltpu.sync_copy(s1_hbm.at[pl.ds(off, _C_CH)], i1_v)
        cpa = pltpu.async_copy(ys_hbm.at[i0_v], bufa, sema)
        cpb = pltpu.async_copy(ys_hbm.at[i1_v], bufb, semb)
        cpa.wait()
        cpb.wait()

        def add_row(r, _):
            for j in range(DH // 16):
                sl = pl.ds(j * 16, 16)
                bufa[r, sl] = bufa[r, sl] + bufb[r, sl]
            return 0

        lax.fori_loop(0, _C_CH, add_row, 0)
        pltpu.sync_copy(bufa, y_hbm.at[pl.ds(off, _C_CH)])


def _sc_combine(ys, src0, src1):
    k = pl.kernel(
        _combine_body,
        out_type=jax.ShapeDtypeStruct((T, DH), jnp.float32),
        mesh=_sc_mesh(),
        scratch_types=[
            pltpu.VMEM((_C_CH,), jnp.int32),
            pltpu.VMEM((_C_CH,), jnp.int32),
            pltpu.VMEM((_C_CH, DH), jnp.float32),
            pltpu.VMEM((_C_CH, DH), jnp.float32),
            pltpu.SemaphoreType.DMA,
            pltpu.SemaphoreType.DMA,
        ],
    )
    return k(ys, src0, src1)


def _router_body(x_ref, wr_ref, a0_ref, a1_ref, w0_ref, w1_ref):
    xb = x_ref[...]
    logits = jax.lax.dot_general(
        xb, wr_ref[...], (((1,), (1,)), ((), ())),
        preferred_element_type=jnp.float32)
    iota = jax.lax.broadcasted_iota(jnp.int32, logits.shape, 1)
    m1 = jnp.max(logits, axis=1, keepdims=True)
    a0 = jnp.min(jnp.where(logits == m1, iota, E), axis=1, keepdims=True)
    masked = jnp.where(iota == a0, -jnp.inf, logits)
    m2 = jnp.max(masked, axis=1, keepdims=True)
    a1 = jnp.min(jnp.where(masked == m2, iota, E), axis=1, keepdims=True)
    r = jnp.exp(m2 - m1)
    w0 = 1.0 / (1.0 + r)
    a0_ref[...] = a0
    a1_ref[...] = a1
    w0_ref[...] = w0
    w1_ref[...] = r * w0


def _router(x, Wr):
    nt = T // BT
    return pl.pallas_call(
        _router_body,
        grid=(nt,),
        in_specs=[
            pl.BlockSpec((BT, DH), lambda i: (i, 0)),
            pl.BlockSpec((E, DH), lambda i: (0, 0)),
        ],
        out_specs=[
            pl.BlockSpec((BT, 1), lambda i: (i, 0)),
            pl.BlockSpec((BT, 1), lambda i: (i, 0)),
            pl.BlockSpec((BT, 1), lambda i: (i, 0)),
            pl.BlockSpec((BT, 1), lambda i: (i, 0)),
        ],
        out_shape=[
            jax.ShapeDtypeStruct((T, 1), jnp.int32),
            jax.ShapeDtypeStruct((T, 1), jnp.int32),
            jax.ShapeDtypeStruct((T, 1), jnp.float32),
            jax.ShapeDtypeStruct((T, 1), jnp.float32),
        ],
    )(x, Wr)


def _ffn_body(be_ref, xs_ref, win_ref, wout_ref, wp_ref, ys_ref):
    xb = xs_ref[...].astype(jnp.bfloat16)
    h = jax.lax.dot_general(
        xb, win_ref[0].astype(jnp.bfloat16), (((1,), (1,)), ((), ())),
        preferred_element_type=jnp.float32)
    h = 0.5 * h * (1.0 + jax.lax.erf(h * 0.7071067811865476))
    out = jax.lax.dot_general(
        h.astype(jnp.bfloat16), wout_ref[0].astype(jnp.bfloat16),
        (((1,), (0,)), ((), ())),
        preferred_element_type=jnp.float32)
    ys_ref[...] = out * wp_ref[...]


def _grouped_ffn(xs, W_in, W_out, w_pad, blk_exp):
    grid_spec = pltpu.PrefetchScalarGridSpec(
        num_scalar_prefetch=1,
        grid=(NB,),
        in_specs=[
            pl.BlockSpec((BM, DH), lambda b, be: (b, 0)),
            pl.BlockSpec((1, DHID, DH), lambda b, be: (be[b], 0, 0)),
            pl.BlockSpec((1, DHID, DH), lambda b, be: (be[b], 0, 0)),
            pl.BlockSpec((BM, 1), lambda b, be: (b, 0)),
        ],
        out_specs=pl.BlockSpec((BM, DH), lambda b, be: (b, 0)),
    )
    return pl.pallas_call(
        _ffn_body,
        grid_spec=grid_spec,
        out_shape=jax.ShapeDtypeStruct((NPAD, DH), jnp.float32),
    )(blk_exp, xs, W_in, W_out, w_pad)


def kernel(x, Wr, W_in, W_out):
    a0, a1, w0, w1 = _router(x, Wr)

    e_flat = jnp.concatenate([a0, a1], axis=1).reshape(-1)
    w_flat = jnp.concatenate([w0, w1], axis=1).reshape(-1)
    t_flat = jnp.arange(T * TOPK, dtype=jnp.int32) // TOPK
    oh = (e_flat[:, None] == jnp.arange(E, dtype=jnp.int32)[None, :])
    csum = jnp.cumsum(oh.astype(jnp.int32), axis=0)
    rank = jnp.take_along_axis(csum, e_flat[:, None], axis=1)[:, 0] - 1
    counts = csum[-1]
    padded = ((counts + BM - 1) // BM) * BM
    ends = jnp.cumsum(padded)
    off = ends - padded
    dst = off[e_flat] + rank

    tok_pad = jnp.zeros((NPAD,), jnp.int32).at[dst].set(
        t_flat, unique_indices=True)
    w_pad = jnp.zeros((NPAD, 1), jnp.float32).at[dst, 0].set(
        w_flat, unique_indices=True)
    blk_starts = jnp.arange(NB, dtype=jnp.int32) * BM
    blk_exp = jnp.minimum(
        jnp.sum((blk_starts[:, None] >= ends[None, :]).astype(jnp.int32),
                axis=1), E - 1)

    xs = _sc_gather(x, tok_pad)

    ys = _grouped_ffn(xs, W_in, W_out, w_pad, blk_exp)

    src = dst.reshape(T, TOPK)
    y = _sc_combine(ys, src[:, 0], src[:, 1])
    return y


# speedup vs baseline: 2.7654x; 1.4782x over previous
import functools

import jax
import jax.numpy as jnp
from jax import lax
from jax.experimental import pallas as pl
from jax.experimental.pallas import tpu as pltpu
from jax.experimental.pallas import tpu_sc as plsc

T, DH, DHID, E, TOPK = 8192, 1024, 2048, 8, 2

BT = 256
BM = 256
NPAD = T * TOPK + E * BM
NB = NPAD // BM

NC, NS = 2, 16
NW = NC * NS


def _sc_mesh():
    return plsc.VectorSubcoreMesh(core_axis_name="c", subcore_axis_name="s")


def _wid():
    return lax.axis_index("s") * NC + lax.axis_index("c")


_D_TOK = T // NW
_D_CH = 16
_D_NCH = _D_TOK // _D_CH


def _dispatch_body(x_hbm, d0_hbm, d1_hbm, xs_hbm,
                   i0_v, i1_v, buf0, buf1, lsem0, lsem1,
                   s0sem0, s0sem1, s1sem0, s1sem1):
    w = _wid()
    base = w * _D_TOK
    pltpu.sync_copy(d0_hbm.at[w], i0_v)
    pltpu.sync_copy(d1_hbm.at[w], i1_v)

    def step(g, _):
        c0 = g * 2
        c1 = c0 + 1
        l0 = pltpu.async_copy(
            x_hbm.at[pl.ds(base + c0 * _D_CH, _D_CH)], buf0, lsem0)
        l1 = pltpu.async_copy(
            x_hbm.at[pl.ds(base + c1 * _D_CH, _D_CH)], buf1, lsem1)
        l0.wait()
        s00 = pltpu.async_copy(buf0, xs_hbm.at[i0_v.at[c0]], s0sem0)
        s01 = pltpu.async_copy(buf0, xs_hbm.at[i1_v.at[c0]], s1sem0)
        l1.wait()
        s10 = pltpu.async_copy(buf1, xs_hbm.at[i0_v.at[c1]], s0sem1)
        s11 = pltpu.async_copy(buf1, xs_hbm.at[i1_v.at[c1]], s1sem1)
        s00.wait()
        s01.wait()
        s10.wait()
        s11.wait()
        return 0

    lax.fori_loop(0, _D_NCH // 2, step, 0)


def _sc_dispatch(x, d0, d1):
    k = pl.kernel(
        _dispatch_body,
        out_type=jax.ShapeDtypeStruct((NPAD, DH), jnp.float32),
        mesh=_sc_mesh(),
        scratch_types=[
            pltpu.VMEM((_D_NCH, _D_CH), jnp.int32),
            pltpu.VMEM((_D_NCH, _D_CH), jnp.int32),
            pltpu.VMEM((_D_CH, DH), jnp.float32),
            pltpu.VMEM((_D_CH, DH), jnp.float32),
            pltpu.SemaphoreType.DMA,
            pltpu.SemaphoreType.DMA,
            pltpu.SemaphoreType.DMA,
            pltpu.SemaphoreType.DMA,
            pltpu.SemaphoreType.DMA,
            pltpu.SemaphoreType.DMA,
        ],
    )
    return k(x, d0, d1)


_C_TOK = T // NW
_C_CH = 8
_C_NCH = _C_TOK // _C_CH


def _combine_body(ys_hbm, s0_hbm, s1_hbm, w0_hbm, w1_hbm, y_hbm,
                  i0_v, i1_v, wa_v, wb_v, bufa0, bufb0, bufa1, bufb1,
                  ga0s, ga1s, gb0s, gb1s, wb0s, wb1s):
    w = _wid()
    base = w * _C_TOK
    pltpu.sync_copy(s0_hbm.at[w], i0_v)
    pltpu.sync_copy(s1_hbm.at[w], i1_v)
    pltpu.sync_copy(w0_hbm.at[pl.ds(base, _C_TOK)], wa_v)
    pltpu.sync_copy(w1_hbm.at[pl.ds(base, _C_TOK)], wb_v)

    def add_chunk(c, ba, bb):
        def row(r, _):
            wr = c * _C_CH + r
            w0v = wa_v[wr]
            w1v = wb_v[wr]
            for j in range(DH // 16):
                sl = pl.ds(j * 16, 16)
                ba[r, sl] = ba[r, sl] * w0v + bb[r, sl] * w1v
            return 0

        lax.fori_loop(0, _C_CH, row, 0)

    def step(g, _):
        c0 = g * 2
        c1 = c0 + 1
        ga0 = pltpu.async_copy(ys_hbm.at[i0_v.at[c0]], bufa0, ga0s)
        gb0 = pltpu.async_copy(ys_hbm.at[i1_v.at[c0]], bufb0, gb0s)
        ga1 = pltpu.async_copy(ys_hbm.at[i0_v.at[c1]], bufa1, ga1s)
        gb1 = pltpu.async_copy(ys_hbm.at[i1_v.at[c1]], bufb1, gb1s)
        ga0.wait()
        gb0.wait()
        add_chunk(c0, bufa0, bufb0)
        w0k = pltpu.async_copy(
            bufa0, y_hbm.at[pl.ds(base + c0 * _C_CH, _C_CH)], wb0s)
        ga1.wait()
        gb1.wait()
        add_chunk(c1, bufa1, bufb1)
        w1k = pltpu.async_copy(
            bufa1, y_hbm.at[pl.ds(base + c1 * _C_CH, _C_CH)], wb1s)
        w0k.wait()
        w1k.wait()
        return 0

    lax.fori_loop(0, _C_NCH // 2, step, 0)


def _sc_combine(ys, src0, src1, w0x, w1x):
    k = pl.kernel(
        _combine_body,
        out_type=jax.ShapeDtypeStruct((T, DH), jnp.float32),
        mesh=_sc_mesh(),
        scratch_types=[
            pltpu.VMEM((_C_NCH, _C_CH), jnp.int32),
            pltpu.VMEM((_C_NCH, _C_CH), jnp.int32),
            pltpu.VMEM((_C_TOK, 16), jnp.float32),
            pltpu.VMEM((_C_TOK, 16), jnp.float32),
            pltpu.VMEM((_C_CH, DH), jnp.float32),
            pltpu.VMEM((_C_CH, DH), jnp.float32),
            pltpu.VMEM((_C_CH, DH), jnp.float32),
            pltpu.VMEM((_C_CH, DH), jnp.float32),
            pltpu.SemaphoreType.DMA,
            pltpu.SemaphoreType.DMA,
            pltpu.SemaphoreType.DMA,
            pltpu.SemaphoreType.DMA,
            pltpu.SemaphoreType.DMA,
            pltpu.SemaphoreType.DMA,
        ],
    )
    return k(ys, src0, src1, w0x, w1x)


def _router_body(x_ref, wr_ref, a0_ref, a1_ref, w0_ref, w1_ref):
    xb = x_ref[...]
    logits = jax.lax.dot_general(
        xb, wr_ref[...], (((1,), (1,)), ((), ())),
        preferred_element_type=jnp.float32)
    iota = jax.lax.broadcasted_iota(jnp.int32, logits.shape, 1)
    m1 = jnp.max(logits, axis=1, keepdims=True)
    a0 = jnp.min(jnp.where(logits == m1, iota, E), axis=1, keepdims=True)
    masked = jnp.where(iota == a0, -jnp.inf, logits)
    m2 = jnp.max(masked, axis=1, keepdims=True)
    a1 = jnp.min(jnp.where(masked == m2, iota, E), axis=1, keepdims=True)
    r = jnp.exp(m2 - m1)
    w0 = 1.0 / (1.0 + r)
    a0_ref[...] = a0
    a1_ref[...] = a1
    w0_ref[...] = w0
    w1_ref[...] = r * w0


def _router(x, Wr):
    nt = T // BT
    return pl.pallas_call(
        _router_body,
        grid=(nt,),
        in_specs=[
            pl.BlockSpec((BT, DH), lambda i: (i, 0)),
            pl.BlockSpec((E, DH), lambda i: (0, 0)),
        ],
        out_specs=[
            pl.BlockSpec((BT, 1), lambda i: (i, 0)),
            pl.BlockSpec((BT, 1), lambda i: (i, 0)),
            pl.BlockSpec((BT, 1), lambda i: (i, 0)),
            pl.BlockSpec((BT, 1), lambda i: (i, 0)),
        ],
        out_shape=[
            jax.ShapeDtypeStruct((T, 1), jnp.int32),
            jax.ShapeDtypeStruct((T, 1), jnp.int32),
            jax.ShapeDtypeStruct((T, 1), jnp.float32),
            jax.ShapeDtypeStruct((T, 1), jnp.float32),
        ],
    )(x, Wr)


def _ffn_body(be_ref, xs_ref, win_ref, wout_ref, ys_ref):
    xb = xs_ref[...].astype(jnp.bfloat16)
    h = jax.lax.dot_general(
        xb, win_ref[0].astype(jnp.bfloat16), (((1,), (1,)), ((), ())),
        preferred_element_type=jnp.float32)
    h = 0.5 * h * (1.0 + jax.lax.erf(h * 0.7071067811865476))
    out = jax.lax.dot_general(
        h.astype(jnp.bfloat16), wout_ref[0].astype(jnp.bfloat16),
        (((1,), (0,)), ((), ())),
        preferred_element_type=jnp.float32)
    ys_ref[...] = out


def _grouped_ffn(xs, W_in, W_out, blk_exp):
    grid_spec = pltpu.PrefetchScalarGridSpec(
        num_scalar_prefetch=1,
        grid=(NB,),
        in_specs=[
            pl.BlockSpec((BM, DH), lambda b, be: (b, 0)),
            pl.BlockSpec((1, DHID, DH), lambda b, be: (be[b], 0, 0)),
            pl.BlockSpec((1, DHID, DH), lambda b, be: (be[b], 0, 0)),
        ],
        out_specs=pl.BlockSpec((BM, DH), lambda b, be: (b, 0)),
    )
    return pl.pallas_call(
        _ffn_body,
        grid_spec=grid_spec,
        out_shape=jax.ShapeDtypeStruct((NPAD, DH), jnp.float32),
    )(blk_exp, xs, W_in, W_out)


def kernel(x, Wr, W_in, W_out):
    a0, a1, w0, w1 = _router(x, Wr)

    e_flat = jnp.concatenate([a0, a1], axis=1).reshape(-1)
    w_flat = jnp.concatenate([w0, w1], axis=1).reshape(-1)
    t_flat = jnp.arange(T * TOPK, dtype=jnp.int32) // TOPK
    oh = (e_flat[:, None] == jnp.arange(E, dtype=jnp.int32)[None, :])
    csum = jnp.cumsum(oh.astype(jnp.int32), axis=0)
    rank = jnp.take_along_axis(csum, e_flat[:, None], axis=1)[:, 0] - 1
    counts = csum[-1]
    padded = ((counts + BM - 1) // BM) * BM
    ends = jnp.cumsum(padded)
    off = ends - padded
    dst = off[e_flat] + rank

    blk_starts = jnp.arange(NB, dtype=jnp.int32) * BM
    blk_exp = jnp.minimum(
        jnp.sum((blk_starts[:, None] >= ends[None, :]).astype(jnp.int32),
                axis=1), E - 1)

    src = dst.reshape(T, TOPK)
    d0 = src[:, 0].reshape(NW, _D_NCH, _D_CH)
    d1 = src[:, 1].reshape(NW, _D_NCH, _D_CH)
    s0 = src[:, 0].reshape(NW, _C_NCH, _C_CH)
    s1 = src[:, 1].reshape(NW, _C_NCH, _C_CH)
    w0x = jnp.broadcast_to(w0, (T, 16))
    w1x = jnp.broadcast_to(w1, (T, 16))

    xs = _sc_dispatch(x, d0, d1)

    ys = _grouped_ffn(xs, W_in, W_out, blk_exp)

    y = _sc_combine(ys, s0, s1, w0x, w1x)
    return y


# speedup vs baseline: 2.9204x; 1.0561x over previous
import functools

import jax
import jax.numpy as jnp
from jax import lax
from jax.experimental import pallas as pl
from jax.experimental.pallas import tpu as pltpu
from jax.experimental.pallas import tpu_sc as plsc

T, DH, DHID, E, TOPK = 8192, 1024, 2048, 8, 2

BT = 256
BM = 256
NPAD = T * TOPK + E * BM
NB = NPAD // BM

NC, NS = 2, 16
NW = NC * NS


def _sc_mesh():
    return plsc.VectorSubcoreMesh(core_axis_name="c", subcore_axis_name="s")


def _wid():
    return lax.axis_index("s") * NC + lax.axis_index("c")


_D_TOK = T // NW
_D_CH = 16
_D_NCH = _D_TOK // _D_CH


def _dispatch_body(x_hbm, d0_hbm, d1_hbm, xs_hbm,
                   i0_v, i1_v, buf0, buf1, lsem0, lsem1,
                   s0sem0, s0sem1, s1sem0, s1sem1):
    w = _wid()
    base = w * _D_TOK
    pltpu.sync_copy(d0_hbm.at[w], i0_v)
    pltpu.sync_copy(d1_hbm.at[w], i1_v)

    def step(g, _):
        c0 = g * 2
        c1 = c0 + 1
        l0 = pltpu.async_copy(
            x_hbm.at[pl.ds(base + c0 * _D_CH, _D_CH)], buf0, lsem0)
        l1 = pltpu.async_copy(
            x_hbm.at[pl.ds(base + c1 * _D_CH, _D_CH)], buf1, lsem1)
        l0.wait()
        s00 = pltpu.async_copy(buf0, xs_hbm.at[i0_v.at[c0]], s0sem0)
        s01 = pltpu.async_copy(buf0, xs_hbm.at[i1_v.at[c0]], s1sem0)
        l1.wait()
        s10 = pltpu.async_copy(buf1, xs_hbm.at[i0_v.at[c1]], s0sem1)
        s11 = pltpu.async_copy(buf1, xs_hbm.at[i1_v.at[c1]], s1sem1)
        s00.wait()
        s01.wait()
        s10.wait()
        s11.wait()
        return 0

    lax.fori_loop(0, _D_NCH // 2, step, 0)


def _sc_dispatch(x, d0, d1):
    k = pl.kernel(
        _dispatch_body,
        out_type=jax.ShapeDtypeStruct((NPAD, DH), jnp.float32),
        mesh=_sc_mesh(),
        scratch_types=[
            pltpu.VMEM((_D_NCH, _D_CH), jnp.int32),
            pltpu.VMEM((_D_NCH, _D_CH), jnp.int32),
            pltpu.VMEM((_D_CH, DH), jnp.float32),
            pltpu.VMEM((_D_CH, DH), jnp.float32),
            pltpu.SemaphoreType.DMA,
            pltpu.SemaphoreType.DMA,
            pltpu.SemaphoreType.DMA,
            pltpu.SemaphoreType.DMA,
            pltpu.SemaphoreType.DMA,
            pltpu.SemaphoreType.DMA,
        ],
    )
    return k(x, d0, d1)


_C_TOK = T // NW
_C_CH = 8
_C_NCH = _C_TOK // _C_CH


def _combine_body(ys_hbm, s0_hbm, s1_hbm, w0_hbm, w1_hbm, y_hbm,
                  i0_v, i1_v, wa_v, wb_v, bufa0, bufb0, bufa1, bufb1,
                  ga0s, ga1s, gb0s, gb1s, wb0s, wb1s):
    w = _wid()
    base = w * _C_TOK
    pltpu.sync_copy(s0_hbm.at[w], i0_v)
    pltpu.sync_copy(s1_hbm.at[w], i1_v)
    pltpu.sync_copy(w0_hbm.at[pl.ds(base, _C_TOK)], wa_v)
    pltpu.sync_copy(w1_hbm.at[pl.ds(base, _C_TOK)], wb_v)

    def add_chunk(c, ba, bb):
        def row(r, _):
            wr = c * _C_CH + r
            w0v = wa_v[wr]
            w1v = wb_v[wr]
            for j in range(DH // 16):
                sl = pl.ds(j * 16, 16)
                ba[r, sl] = ba[r, sl] * w0v + bb[r, sl] * w1v
            return 0

        lax.fori_loop(0, _C_CH, row, 0)

    def step(g, _):
        c0 = g * 2
        c1 = c0 + 1
        ga0 = pltpu.async_copy(ys_hbm.at[i0_v.at[c0]], bufa0, ga0s)
        gb0 = pltpu.async_copy(ys_hbm.at[i1_v.at[c0]], bufb0, gb0s)
        ga1 = pltpu.async_copy(ys_hbm.at[i0_v.at[c1]], bufa1, ga1s)
        gb1 = pltpu.async_copy(ys_hbm.at[i1_v.at[c1]], bufb1, gb1s)
        ga0.wait()
        gb0.wait()
        add_chunk(c0, bufa0, bufb0)
        w0k = pltpu.async_copy(
            bufa0, y_hbm.at[pl.ds(base + c0 * _C_CH, _C_CH)], wb0s)
        ga1.wait()
        gb1.wait()
        add_chunk(c1, bufa1, bufb1)
        w1k = pltpu.async_copy(
            bufa1, y_hbm.at[pl.ds(base + c1 * _C_CH, _C_CH)], wb1s)
        w0k.wait()
        w1k.wait()
        return 0

    lax.fori_loop(0, _C_NCH // 2, step, 0)


def _sc_combine(ys, src0, src1, w0x, w1x):
    k = pl.kernel(
        _combine_body,
        out_type=jax.ShapeDtypeStruct((T, DH), jnp.float32),
        mesh=_sc_mesh(),
        scratch_types=[
            pltpu.VMEM((_C_NCH, _C_CH), jnp.int32),
            pltpu.VMEM((_C_NCH, _C_CH), jnp.int32),
            pltpu.VMEM((_C_TOK, 16), jnp.float32),
            pltpu.VMEM((_C_TOK, 16), jnp.float32),
            pltpu.VMEM((_C_CH, DH), jnp.float32),
            pltpu.VMEM((_C_CH, DH), jnp.float32),
            pltpu.VMEM((_C_CH, DH), jnp.float32),
            pltpu.VMEM((_C_CH, DH), jnp.float32),
            pltpu.SemaphoreType.DMA,
            pltpu.SemaphoreType.DMA,
            pltpu.SemaphoreType.DMA,
            pltpu.SemaphoreType.DMA,
            pltpu.SemaphoreType.DMA,
            pltpu.SemaphoreType.DMA,
        ],
    )
    return k(ys, src0, src1, w0x, w1x)


def _router_body(x_ref, wr_ref, a0_ref, a1_ref, w0_ref, w1_ref,
                 r0_ref, r1_ref, cnt_ref, base_scr):
    @pl.when(pl.program_id(0) == 0)
    def _init():
        base_scr[...] = jnp.zeros((1, E), jnp.float32)

    xb = x_ref[...]
    logits = jax.lax.dot_general(
        xb, wr_ref[...], (((1,), (1,)), ((), ())),
        preferred_element_type=jnp.float32)
    iota = jax.lax.broadcasted_iota(jnp.int32, logits.shape, 1)
    m1 = jnp.max(logits, axis=1, keepdims=True)
    a0 = jnp.min(jnp.where(logits == m1, iota, E), axis=1, keepdims=True)
    masked = jnp.where(iota == a0, -jnp.inf, logits)
    m2 = jnp.max(masked, axis=1, keepdims=True)
    a1 = jnp.min(jnp.where(masked == m2, iota, E), axis=1, keepdims=True)
    r = jnp.exp(m2 - m1)
    w0 = 1.0 / (1.0 + r)
    a0_ref[...] = a0
    a1_ref[...] = a1
    w0_ref[...] = w0
    w1_ref[...] = r * w0

    oh0 = (iota == a0).astype(jnp.float32)
    oh1 = (iota == a1).astype(jnp.float32)
    ohs = oh0 + oh1
    ri = jax.lax.broadcasted_iota(jnp.int32, (BT, BT), 0)
    ci = jax.lax.broadcasted_iota(jnp.int32, (BT, BT), 1)
    lt = (ci < ri).astype(jnp.float32)
    base = base_scr[...]
    cum = jax.lax.dot_general(
        lt, ohs, (((1,), (0,)), ((), ())),
        preferred_element_type=jnp.float32) + base
    r0_ref[...] = jnp.sum(cum * oh0, axis=1, keepdims=True).astype(jnp.int32)
    r1_ref[...] = jnp.sum(cum * oh1, axis=1, keepdims=True).astype(jnp.int32)
    base = base + jnp.sum(ohs, axis=0, keepdims=True)
    base_scr[...] = base
    cnt_ref[...] = base.astype(jnp.int32)


def _router(x, Wr):
    nt = T // BT
    col = pl.BlockSpec((BT, 1), lambda i: (i, 0))
    return pl.pallas_call(
        _router_body,
        grid=(nt,),
        in_specs=[
            pl.BlockSpec((BT, DH), lambda i: (i, 0)),
            pl.BlockSpec((E, DH), lambda i: (0, 0)),
        ],
        out_specs=[col, col, col, col, col, col,
                   pl.BlockSpec((1, E), lambda i: (0, 0))],
        out_shape=[
            jax.ShapeDtypeStruct((T, 1), jnp.int32),
            jax.ShapeDtypeStruct((T, 1), jnp.int32),
            jax.ShapeDtypeStruct((T, 1), jnp.float32),
            jax.ShapeDtypeStruct((T, 1), jnp.float32),
            jax.ShapeDtypeStruct((T, 1), jnp.int32),
            jax.ShapeDtypeStruct((T, 1), jnp.int32),
            jax.ShapeDtypeStruct((1, E), jnp.int32),
        ],
        scratch_shapes=[pltpu.VMEM((1, E), jnp.float32)],
    )(x, Wr)


def _ffn_body(be_ref, xs_ref, win_ref, wout_ref, ys_ref):
    xb = xs_ref[...].astype(jnp.bfloat16)
    h = jax.lax.dot_general(
        xb, win_ref[0].astype(jnp.bfloat16), (((1,), (1,)), ((), ())),
        preferred_element_type=jnp.float32)
    h = 0.5 * h * (1.0 + jax.lax.erf(h * 0.7071067811865476))
    out = jax.lax.dot_general(
        h.astype(jnp.bfloat16), wout_ref[0].astype(jnp.bfloat16),
        (((1,), (0,)), ((), ())),
        preferred_element_type=jnp.float32)
    ys_ref[...] = out


def _grouped_ffn(xs, W_in, W_out, blk_exp):
    grid_spec = pltpu.PrefetchScalarGridSpec(
        num_scalar_prefetch=1,
        grid=(NB,),
        in_specs=[
            pl.BlockSpec((BM, DH), lambda b, be: (b, 0)),
            pl.BlockSpec((1, DHID, DH), lambda b, be: (be[b], 0, 0)),
            pl.BlockSpec((1, DHID, DH), lambda b, be: (be[b], 0, 0)),
        ],
        out_specs=pl.BlockSpec((BM, DH), lambda b, be: (b, 0)),
    )
    return pl.pallas_call(
        _ffn_body,
        grid_spec=grid_spec,
        out_shape=jax.ShapeDtypeStruct((NPAD, DH), jnp.float32),
    )(blk_exp, xs, W_in, W_out)


def kernel(x, Wr, W_in, W_out):
    a0, a1, w0, w1, r0, r1, cnt = _router(x, Wr)

    counts = cnt[0]
    padded = ((counts + BM - 1) // BM) * BM
    ends = jnp.cumsum(padded)
    off = ends - padded
    dst0 = jnp.take(off, a0[:, 0]) + r0[:, 0]
    dst1 = jnp.take(off, a1[:, 0]) + r1[:, 0]

    blk_starts = jnp.arange(NB, dtype=jnp.int32) * BM
    blk_exp = jnp.minimum(
        jnp.sum((blk_starts[:, None] >= ends[None, :]).astype(jnp.int32),
                axis=1), E - 1)

    d0 = dst0.reshape(NW, _D_NCH, _D_CH)
    d1 = dst1.reshape(NW, _D_NCH, _D_CH)
    s0 = dst0.reshape(NW, _C_NCH, _C_CH)
    s1 = dst1.reshape(NW, _C_NCH, _C_CH)
    w0x = jnp.broadcast_to(w0, (T, 16))
    w1x = jnp.broadcast_to(w1, (T, 16))

    xs = _sc_dispatch(x, d0, d1)

    ys = _grouped_ffn(xs, W_in, W_out, blk_exp)

    y = _sc_combine(ys, s0, s1, w0x, w1x)
    return y
